# baseline (speedup 1.0000x reference)
import jax
import jax.numpy as jnp
from jax import lax
from jax.experimental import pallas as pl
from jax.experimental.pallas import tpu as pltpu
from jax.experimental.pallas import tpu_sc as plsc

_NUM_FIELDS = 26
_FIELD_SIZE = 100000
_BATCH = 16384
_D = 16
_N = _BATCH * _NUM_FIELDS
_L = 16

_NC, _NS = 2, 16
_NW = _NC * _NS
_PER_W = _N // _NW
_GROW = 128
_KG = 13
_CH = _KG * _GROW
_NSUB = _PER_W // _CH
_NVEC = _CH // _L


def _body(x_hbm, table_hbm, out_hbm, offs_v, idx_v, rows_v, sem):
    wid = lax.axis_index("s") * _NC + lax.axis_index("c")
    base = wid * _PER_W

    iota = lax.iota(jnp.int32, _L)

    def fill_vec(v, _):
        offs_v[pl.ds(v * _L, _L)] = ((v * _L + iota) % _NUM_FIELDS) * _FIELD_SIZE
        return 0

    lax.fori_loop(0, _NVEC, fill_vec, 0)

    def sub_chunk(s, _):
        sub = base + s * _CH
        pltpu.sync_copy(x_hbm.at[pl.ds(sub, _CH)], idx_v)

        def add_vec(v, _):
            sl = pl.ds(v * _L, _L)
            idx_v[sl] = idx_v[sl] + offs_v[sl]
            return 0

        lax.fori_loop(0, _NVEC, add_vec, 0)

        descs = []
        for r in range(_KG):
            descs.append(
                pltpu.async_copy(
                    table_hbm.at[idx_v.at[pl.ds(r * _GROW, _GROW)]],
                    rows_v.at[pl.ds(r * _GROW, _GROW)],
                    sem,
                )
            )
        for d in descs:
            d.wait()

        pltpu.sync_copy(rows_v, out_hbm.at[pl.ds(sub, _CH)])
        return 0

    lax.fori_loop(0, _NSUB, sub_chunk, 0)


@jax.jit
def kernel(x, table):
    x_flat = x.reshape(_N)
    mesh = plsc.VectorSubcoreMesh(core_axis_name="c", subcore_axis_name="s")
    out = pl.kernel(
        _body,
        out_type=jax.ShapeDtypeStruct((_N, _D), jnp.float32),
        mesh=mesh,
        compiler_params=pltpu.CompilerParams(use_tc_tiling_on_sc=False),
        scratch_types=[
            pltpu.VMEM((_CH,), jnp.int32),
            pltpu.VMEM((_CH,), jnp.int32),
            pltpu.VMEM((_CH, _D), jnp.float32),
            pltpu.SemaphoreType.DMA,
        ],
    )(x_flat, table)
    return out.reshape(_BATCH, _NUM_FIELDS, _D)


# speedup vs baseline: 1.2263x; 1.2263x over previous
import jax
import jax.numpy as jnp
from jax import lax
from jax.experimental import pallas as pl
from jax.experimental.pallas import tpu as pltpu
from jax.experimental.pallas import tpu_sc as plsc

_NUM_FIELDS = 26
_FIELD_SIZE = 100000
_BATCH = 16384
_D = 16
_N = _BATCH * _NUM_FIELDS
_L = 16

_NC, _NS = 2, 16
_NW = _NC * _NS
_BT = 128
_NBT = _BATCH // _BT
_BT_PER_W = _NBT // _NW
_CH = _BT * _NUM_FIELDS
_GROW = 128
_KG = _CH // _GROW
_NVEC = _CH // _L
_BLK = _NUM_FIELDS * _D * _BT
_OUT_FLAT = _NUM_FIELDS * 2 * _NBT * 8 * _BT


def _body(x_hbm, table_hbm, out_hbm, offs_v, idx_v, rows_v, blk_v, sem):
    wid = lax.axis_index("s") * _NC + lax.axis_index("c")

    iota = lax.iota(jnp.int32, _L)
    iota26 = iota * _NUM_FIELDS

    def fill_vec(v, _):
        offs_v[pl.ds(v * _L, _L)] = ((v * _L + iota) % _NUM_FIELDS) * _FIELD_SIZE
        return 0

    lax.fori_loop(0, _NVEC, fill_vec, 0)

    def batch_tile(bi, _):
        bt = wid * _BT_PER_W + bi
        sub = bt * _CH
        pltpu.sync_copy(x_hbm.at[pl.ds(sub, _CH)], idx_v)

        def add_vec(v, _):
            sl = pl.ds(v * _L, _L)
            idx_v[sl] = idx_v[sl] + offs_v[sl]
            return 0

        lax.fori_loop(0, _NVEC, add_vec, 0)

        descs = []
        for r in range(_KG):
            descs.append(
                pltpu.async_copy(
                    table_hbm.at[idx_v.at[pl.ds(r * _GROW, _GROW)]],
                    rows_v.at[pl.ds(r * _GROW, _GROW)],
                    sem,
                )
            )
        for dsc in descs:
            dsc.wait()

        def reorder_f(f, _):
            def reorder_d(d, _):
                dvec = lax.broadcast(d, (_L,))
                dst_base = f * (_D * _BT) + d * _BT

                for g in range(_BT // _L):
                    rvec = iota26 + (g * _L * _NUM_FIELDS + f)
                    val = plsc.load_gather(rows_v, [rvec, dvec])
                    blk_v[pl.ds(dst_base + g * _L, _L)] = val
                return 0

            lax.fori_loop(0, _D, reorder_d, 0)
            return 0

        lax.fori_loop(0, _NUM_FIELDS, reorder_f, 0)

        odescs = []
        for f in range(_NUM_FIELDS):
            for dh in range(2):
                src = (f * 2 + dh) * 1024
                odescs.append(
                    pltpu.async_copy(
                        blk_v.at[pl.ds(src, 1024)],
                        out_hbm.at[
                            pl.ds(((f * 2 + dh) * _NBT + bt) * 1024, 1024)
                        ],
                        sem,
                    )
                )
        for dsc in odescs:
            dsc.wait()
        return 0

    lax.fori_loop(0, _BT_PER_W, batch_tile, 0)


@jax.jit
def kernel(x, table):
    x_flat = x.reshape(_N)
    mesh = plsc.VectorSubcoreMesh(core_axis_name="c", subcore_axis_name="s")
    out_flat = pl.kernel(
        _body,
        out_type=jax.ShapeDtypeStruct((_OUT_FLAT,), jnp.float32),
        mesh=mesh,
        compiler_params=pltpu.CompilerParams(
            use_tc_tiling_on_sc=False, needs_layout_passes=False
        ),
        scratch_types=[
            pltpu.VMEM((_CH,), jnp.int32),
            pltpu.VMEM((_CH,), jnp.int32),
            pltpu.VMEM((_CH, _D), jnp.float32),
            pltpu.VMEM((_BLK,), jnp.float32),
            pltpu.SemaphoreType.DMA,
        ],
    )(x_flat, table)
    out5 = out_flat.reshape(_NUM_FIELDS, 2, _NBT, 8, _BT)
    return out5.transpose(2, 4, 0, 1, 3).reshape(_BATCH, _NUM_FIELDS, _D)


# speedup vs baseline: 2.1071x; 1.7182x over previous
import jax
import jax.numpy as jnp
from jax import lax
from jax.experimental import pallas as pl
from jax.experimental.pallas import tpu as pltpu
from jax.experimental.pallas import tpu_sc as plsc

_NUM_FIELDS = 26
_FIELD_SIZE = 100000
_BATCH = 16384
_D = 16
_N = _BATCH * _NUM_FIELDS
_L = 16

_NC, _NS = 2, 16
_NW = _NC * _NS
_BT = 128
_NBT = _BATCH // _BT
_BT_PER_W = _NBT // _NW
_CH = _BT * _NUM_FIELDS
_GROW = 128
_KG = _CH // _GROW
_NVEC = _CH // _L
_BLK = _NUM_FIELDS * _D * _BT
_OUT_FLAT = _NUM_FIELDS * 2 * _NBT * 8 * _BT


def _body(x_hbm, table_hbm, out_hbm, offs_v, idx_v, rows_v, blk_v, sem):
    wid = lax.axis_index("s") * _NC + lax.axis_index("c")

    iota = lax.iota(jnp.int32, _L)
    iota26 = iota * _NUM_FIELDS

    def fill_vec(v, _):
        offs_v[pl.ds(v * _L, _L)] = ((v * _L + iota) % _NUM_FIELDS) * _FIELD_SIZE
        return 0

    lax.fori_loop(0, _NVEC, fill_vec, 0)

    def batch_tile(bi, _):
        bt = wid * _BT_PER_W + bi
        sub = bt * _CH
        pltpu.sync_copy(x_hbm.at[pl.ds(sub, _CH)], idx_v)

        def add_vec(v, _):
            sl = pl.ds(v * _L, _L)
            idx_v[sl] = idx_v[sl] + offs_v[sl]
            return 0

        lax.fori_loop(0, _NVEC, add_vec, 0)

        descs = []
        for r in range(_KG):
            descs.append(
                pltpu.async_copy(
                    table_hbm.at[idx_v.at[pl.ds(r * _GROW, _GROW)]],
                    rows_v.at[pl.ds(r * _GROW, _GROW)],
                    sem,
                )
            )
        for dsc in descs:
            dsc.wait()

        def reorder_f(f, _):
            def reorder_d(d, _):
                dvec = lax.broadcast(d, (_L,))
                dst_base = f * (_D * _BT) + d * _BT

                for g in range(_BT // _L):
                    rvec = iota26 + (g * _L * _NUM_FIELDS + f)
                    val = plsc.load_gather(rows_v, [rvec, dvec])
                    blk_v[pl.ds(dst_base + g * _L, _L)] = val
                return 0

            lax.fori_loop(0, _D, reorder_d, 0)
            return 0

        lax.fori_loop(0, _NUM_FIELDS, reorder_f, 0)

        odescs = []
        for f in range(_NUM_FIELDS):
            for dh in range(2):
                src = (f * 2 + dh) * 1024
                odescs.append(
                    pltpu.async_copy(
                        blk_v.at[pl.ds(src, 1024)],
                        out_hbm.at[
                            pl.ds(((f * 2 + dh) * _NBT + bt) * 1024, 1024)
                        ],
                        sem,
                    )
                )
        for dsc in odescs:
            dsc.wait()
        return 0

    lax.fori_loop(0, _BT_PER_W, batch_tile, 0)


_V = 2600000
_SB = 2048
_NSB = (_V + _SB - 1) // _SB
_TAILC = _V - (_NSB - 1) * _SB
_SB_PER_W_HI = 40
_W_HI = _NSB - 32 * 39


def _tbody(tbl_t, out_lin, vbuf, rowbuf):
    wid = lax.axis_index("s") * _NC + lax.axis_index("c")
    iota = lax.iota(jnp.int32, _L)
    iota16 = iota * _D

    start = wid * _SB_PER_W_HI - lax.max(0, wid - _W_HI)
    count = _SB_PER_W_HI - jnp.where(wid >= _W_HI, 1, 0)

    def do_block(c0, rcols, wcols):
        pltpu.sync_copy(
            tbl_t.at[pl.ds(0, 8), pl.ds(c0, rcols)],
            vbuf.at[pl.ds(0, 8), pl.ds(0, rcols)],
        )
        pltpu.sync_copy(
            tbl_t.at[pl.ds(8, 8), pl.ds(c0, rcols)],
            vbuf.at[pl.ds(8, 8), pl.ds(0, rcols)],
        )

        def tr_r(r, _):
            def tr_g(g, _):
                v16 = vbuf[r, pl.ds(g * _L, _L)]
                plsc.store_scatter(rowbuf, [iota16 + (g * 256 + r)], v16)
                return 0

            lax.fori_loop(0, wcols // _L, tr_g, 0)
            return 0

        lax.fori_loop(0, _D, tr_r, 0)
        pltpu.sync_copy(
            rowbuf.at[pl.ds(0, wcols * _D)], out_lin.at[pl.ds(c0 * _D, wcols * _D)]
        )

    def sblock(i, _):
        sb = start + i
        c0 = sb * _SB

        @pl.when(sb < _NSB - 1)
        def _full():
            do_block(c0, _SB, _SB)

        @pl.when(sb == _NSB - 1)
        def _tail():
            do_block(c0, 9 * 128, _TAILC)

        return 0

    lax.fori_loop(0, count, sblock, 0)


def _relayout_table(table):
    mesh = plsc.VectorSubcoreMesh(core_axis_name="c", subcore_axis_name="s")
    return pl.kernel(
        _tbody,
        out_type=jax.ShapeDtypeStruct((_V * _D,), jnp.float32),
        mesh=mesh,
        compiler_params=pltpu.CompilerParams(
            use_tc_tiling_on_sc=True,
            needs_layout_passes=False,
            disable_bounds_checks=True,
        ),
        scratch_types=[
            pltpu.VMEM((_D, _SB), jnp.float32),
            pltpu.VMEM((_SB * _D,), jnp.float32),
        ],
    )(table.T)


@jax.jit
def kernel(x, table):
    x_flat = x.reshape(_N)
    table_lin = _relayout_table(table).reshape(_V, _D)
    mesh = plsc.VectorSubcoreMesh(core_axis_name="c", subcore_axis_name="s")
    out_flat = pl.kernel(
        _body,
        out_type=jax.ShapeDtypeStruct((_OUT_FLAT,), jnp.float32),
        mesh=mesh,
        compiler_params=pltpu.CompilerParams(
            use_tc_tiling_on_sc=False, needs_layout_passes=False
        ),
        scratch_types=[
            pltpu.VMEM((_CH,), jnp.int32),
            pltpu.VMEM((_CH,), jnp.int32),
            pltpu.VMEM((_CH, _D), jnp.float32),
            pltpu.VMEM((_BLK,), jnp.float32),
            pltpu.SemaphoreType.DMA,
        ],
    )(x_flat, table_lin)
    out5 = out_flat.reshape(_NUM_FIELDS, 2, _NBT, 8, _BT)
    return out5.transpose(2, 4, 0, 1, 3).reshape(_BATCH, _NUM_FIELDS, _D)


# speedup vs baseline: 2.9504x; 1.4002x over previous
import jax
import jax.numpy as jnp
from jax import lax
from jax.experimental import pallas as pl
from jax.experimental.pallas import tpu as pltpu
from jax.experimental.pallas import tpu_sc as plsc

_NUM_FIELDS = 26
_FIELD_SIZE = 100000
_BATCH = 16384
_D = 16
_N = _BATCH * _NUM_FIELDS
_L = 16

_NC, _NS = 2, 16
_NW = _NC * _NS

_V = 2600000
_SB = 1024
_NFULL = _V // _SB
_TAILC = _V - _NFULL * _SB
_PHA = 79
_PHB0 = _NW * _PHA
_NPHB = _NFULL + 1 - _PHB0


def _tbody(tbl_t, out_lin, va, vb, rowbuf, sem_ia, sem_ib, sem_o):
    wid = lax.axis_index("s") * _NC + lax.axis_index("c")
    iota = lax.iota(jnp.int32, _L)
    iota16 = iota * _D

    def fire_in(sb, vbuf, sem):
        c0 = sb * _SB
        pltpu.async_copy(
            tbl_t.at[pl.ds(0, 8), pl.ds(c0, _SB)], vbuf.at[pl.ds(0, 8)], sem
        )
        pltpu.async_copy(
            tbl_t.at[pl.ds(8, 8), pl.ds(c0, _SB)], vbuf.at[pl.ds(8, 8)], sem
        )

    def wait_in(vbuf, sem):
        for h in range(2):
            pltpu.make_async_copy(
                tbl_t.at[pl.ds(h * 8, 8), pl.ds(0, _SB)],
                vbuf.at[pl.ds(h * 8, 8)],
                sem,
            ).wait()

    def transpose(vbuf, ng):
        def tg(g, _):
            base = iota16 + g * (_L * _D)
            for r in range(_D):
                v16 = vbuf[r, pl.ds(g * _L, _L)]
                plsc.store_scatter(rowbuf, [base + r], v16)
            return 0

        lax.fori_loop(0, ng, tg, 0)

    def fire_out(sb, n):
        pltpu.async_copy(
            rowbuf.at[pl.ds(0, n)], out_lin.at[pl.ds(sb * (_SB * _D), n)], sem_o
        )

    def wait_out(n):
        pltpu.make_async_copy(
            rowbuf.at[pl.ds(0, n)], out_lin.at[pl.ds(0, n)], sem_o
        ).wait()

    base = wid * _PHA
    fire_in(base, va, sem_ia)

    def pair(k, _):
        sb0 = base + 2 * k
        fire_in(sb0 + 1, vb, sem_ib)
        wait_in(va, sem_ia)

        @pl.when(k > 0)
        def _():
            wait_out(_SB * _D)

        transpose(va, _SB // _L)
        fire_out(sb0, _SB * _D)
        fire_in(sb0 + 2, va, sem_ia)
        wait_in(vb, sem_ib)
        wait_out(_SB * _D)
        transpose(vb, _SB // _L)
        fire_out(sb0 + 1, _SB * _D)
        return 0

    lax.fori_loop(0, (_PHA - 1) // 2, pair, 0)
    wait_in(va, sem_ia)
    wait_out(_SB * _D)
    transpose(va, _SB // _L)
    fire_out(base + _PHA - 1, _SB * _D)
    wait_out(_SB * _D)

    @pl.when(wid < _NPHB)
    def _phase_b():
        sb = _PHB0 + wid
        fire_in(sb, va, sem_ia)
        wait_in(va, sem_ia)

        @pl.when(sb < _NFULL)
        def _full():
            transpose(va, _SB // _L)
            fire_out(sb, _SB * _D)
            wait_out(_SB * _D)

        @pl.when(sb == _NFULL)
        def _tail():
            transpose(va, _TAILC // _L)
            fire_out(sb, _TAILC * _D)
            wait_out(_TAILC * _D)


def _relayout_table(table):
    mesh = plsc.VectorSubcoreMesh(core_axis_name="c", subcore_axis_name="s")
    return pl.kernel(
        _tbody,
        out_type=jax.ShapeDtypeStruct((_V * _D,), jnp.float32),
        mesh=mesh,
        compiler_params=pltpu.CompilerParams(
            use_tc_tiling_on_sc=True,
            needs_layout_passes=False,
            disable_bounds_checks=True,
        ),
        scratch_types=[
            pltpu.VMEM((_D, _SB), jnp.float32),
            pltpu.VMEM((_D, _SB), jnp.float32),
            pltpu.VMEM((_SB * _D,), jnp.float32),
            pltpu.SemaphoreType.DMA,
            pltpu.SemaphoreType.DMA,
            pltpu.SemaphoreType.DMA,
        ],
    )(table.T)


_BT = 128
_NBT = _BATCH // _BT
_BT_PER_W = _NBT // _NW
_CH = _BT * _NUM_FIELDS
_GROW = 128
_KG = _CH // _GROW
_NVEC = _CH // _L
_BLK = _NUM_FIELDS * _D * _BT
_OUT_FLAT = _NUM_FIELDS * 2 * _NBT * 8 * _BT


def _body(x_hbm, table_hbm, out_hbm, offs_v, idx_v, rows_v, blk_v, sem):
    wid = lax.axis_index("s") * _NC + lax.axis_index("c")

    iota = lax.iota(jnp.int32, _L)
    iota26 = iota * _NUM_FIELDS
    dvecs = [lax.broadcast(jnp.int32(d), (_L,)) for d in range(_D)]

    def fill_vec(v, _):
        offs_v[pl.ds(v * _L, _L)] = ((v * _L + iota) % _NUM_FIELDS) * _FIELD_SIZE
        return 0

    lax.fori_loop(0, _NVEC, fill_vec, 0)

    def batch_tile(bi, _):
        bt = wid * _BT_PER_W + bi
        sub = bt * _CH
        pltpu.sync_copy(x_hbm.at[pl.ds(sub, _CH)], idx_v)

        def add_vec(v, _):
            sl = pl.ds(v * _L, _L)
            idx_v[sl] = idx_v[sl] + offs_v[sl]
            return 0

        lax.fori_loop(0, _NVEC, add_vec, 0)

        descs = []
        for r in range(_KG):
            descs.append(
                pltpu.async_copy(
                    table_hbm.at[idx_v.at[pl.ds(r * _GROW, _GROW)]],
                    rows_v.at[pl.ds(r * _GROW, _GROW)],
                    sem,
                )
            )
        for dsc in descs:
            dsc.wait()

        def reorder_f(f, _):
            fvec = iota26 + f
            rvecs = [fvec + g * (_L * _NUM_FIELDS) for g in range(_BT // _L)]
            fbase = f * (_D * _BT)
            for d in range(_D):
                for g in range(_BT // _L):
                    val = plsc.load_gather(rows_v, [rvecs[g], dvecs[d]])
                    blk_v[pl.ds(fbase + d * _BT + g * _L, _L)] = val
            return 0

        lax.fori_loop(0, _NUM_FIELDS, reorder_f, 0)

        odescs = []
        for f in range(_NUM_FIELDS):
            for dh in range(2):
                src = (f * 2 + dh) * 1024
                odescs.append(
                    pltpu.async_copy(
                        blk_v.at[pl.ds(src, 1024)],
                        out_hbm.at[
                            pl.ds(((f * 2 + dh) * _NBT + bt) * 1024, 1024)
                        ],
                        sem,
                    )
                )
        for dsc in odescs:
            dsc.wait()
        return 0

    lax.fori_loop(0, _BT_PER_W, batch_tile, 0)


@jax.jit
def kernel(x, table):
    x_flat = x.reshape(_N)
    table_lin = _relayout_table(table).reshape(_V, _D)
    mesh = plsc.VectorSubcoreMesh(core_axis_name="c", subcore_axis_name="s")
    out_flat = pl.kernel(
        _body,
        out_type=jax.ShapeDtypeStruct((_OUT_FLAT,), jnp.float32),
        mesh=mesh,
        compiler_params=pltpu.CompilerParams(
            use_tc_tiling_on_sc=False, needs_layout_passes=False
        ),
        scratch_types=[
            pltpu.VMEM((_CH,), jnp.int32),
            pltpu.VMEM((_CH,), jnp.int32),
            pltpu.VMEM((_CH, _D), jnp.float32),
            pltpu.VMEM((_BLK,), jnp.float32),
            pltpu.SemaphoreType.DMA,
        ],
    )(x_flat, table_lin)
    out5 = out_flat.reshape(_NUM_FIELDS, 2, _NBT, 8, _BT)
    return out5.transpose(2, 4, 0, 1, 3).reshape(_BATCH, _NUM_FIELDS, _D)
